# baseline (speedup 1.0000x reference)
import functools

import jax
import jax.numpy as jnp
from jax import lax
from jax.experimental import pallas as pl
from jax.experimental.pallas import tpu as pltpu
from jax.experimental.pallas import tpu_sc as plsc

R = 64
N = 8192

_info = plsc.get_sparse_core_info()
_NC, _NS, _L = _info.num_cores, _info.num_subcores, _info.num_lanes
_NW = _NC * _NS
_CPW = N // _NW
_NB = 2
_CW = _CPW // _NB
_GPC = _CW // _L


@functools.partial(
    pl.kernel,
    mesh=plsc.VectorSubcoreMesh(core_axis_name="c", subcore_axis_name="s"),
    out_type=(
        jax.ShapeDtypeStruct((1, N), jnp.float32),
        jax.ShapeDtypeStruct((1, N), jnp.int32),
    ),
    scratch_types=[
        pltpu.VMEM((R, _CPW), jnp.float32),
        pltpu.VMEM((_CPW,), jnp.float32),
        pltpu.VMEM((_CPW,), jnp.int32),
    ] + [pltpu.SemaphoreType.DMA] * _NB,
)
def _colmax(x_hbm, vals_hbm, idx_hbm, x_v, mv_v, mi_v, *sems):
    wid = lax.axis_index("s") * _NC + lax.axis_index("c")
    base = wid * _CPW

    copies = [
        pltpu.async_copy(
            x_hbm.at[:, pl.ds(base + c * _CW, _CW)],
            x_v.at[:, pl.ds(c * _CW, _CW)],
            sems[c],
        )
        for c in range(_NB)
    ]

    for c in range(_NB):
        copies[c].wait()

        def group(g, carry):
            cols = pl.ds(c * _CW + g * _L, _L)
            m = x_v[0, cols]
            idx = jnp.zeros((_L,), jnp.int32)
            for r in range(1, R):
                v = x_v[r, cols]
                pred = v > m
                m = jnp.where(pred, v, m)
                idx = jnp.where(pred, jnp.full((_L,), r, jnp.int32), idx)
            mv_v[cols] = m
            mi_v[cols] = idx
            return carry

        lax.fori_loop(0, _GPC, group, 0)

    pltpu.sync_copy(mv_v, vals_hbm.at[0, pl.ds(base, _CPW)])
    pltpu.sync_copy(mi_v, idx_hbm.at[0, pl.ds(base, _CPW)])


def kernel(x):
    vals, idx = _colmax(x)
    return vals, idx.astype(jnp.int64)


# speedup vs baseline: 1.0770x; 1.0770x over previous
import functools

import jax
import jax.numpy as jnp
from jax import lax
from jax.experimental import pallas as pl
from jax.experimental.pallas import tpu as pltpu
from jax.experimental.pallas import tpu_sc as plsc

R = 64
N = 8192

_info = plsc.get_sparse_core_info()
_NC, _NS, _L = _info.num_cores, _info.num_subcores, _info.num_lanes
_NW = _NC * _NS
_CPW = N // _NW
_NB = 2
_CW = _CPW // _NB
_GPC = _CW // _L


@functools.partial(
    pl.kernel,
    mesh=plsc.VectorSubcoreMesh(core_axis_name="c", subcore_axis_name="s"),
    out_type=(
        jax.ShapeDtypeStruct((1, N), jnp.float32),
        jax.ShapeDtypeStruct((1, N), jnp.int32),
    ),
    scratch_types=[
        pltpu.VMEM((R, _CPW), jnp.float32),
        pltpu.VMEM((_CPW,), jnp.float32),
        pltpu.VMEM((_CPW,), jnp.int32),
    ] + [pltpu.SemaphoreType.DMA] * _NB,
)
def _colmax(x_hbm, vals_hbm, idx_hbm, x_v, mv_v, mi_v, *sems):
    wid = lax.axis_index("s") * _NC + lax.axis_index("c")
    base = wid * _CPW

    copies = [
        pltpu.async_copy(
            x_hbm.at[:, pl.ds(base + c * _CW, _CW)],
            x_v.at[:, pl.ds(c * _CW, _CW)],
            sems[c],
        )
        for c in range(_NB)
    ]

    copies[0].wait()

    def group(g, carry):
        for c in range(1, _NB):
            @pl.when(g == c * _GPC)
            def _():
                copies[c].wait()

        cols = pl.ds(g * _L, _L)
        m = x_v[0, cols]
        idx = jnp.zeros((_L,), jnp.int32)
        for r in range(1, R):
            v = x_v[r, cols]
            pred = v > m
            m = jnp.where(pred, v, m)
            idx = jnp.where(pred, jnp.full((_L,), r, jnp.int32), idx)
        mv_v[cols] = m
        mi_v[cols] = idx
        return carry

    lax.fori_loop(0, _NB * _GPC, group, 0)

    pltpu.sync_copy(mv_v, vals_hbm.at[0, pl.ds(base, _CPW)])
    pltpu.sync_copy(mi_v, idx_hbm.at[0, pl.ds(base, _CPW)])


def kernel(x):
    vals, idx = _colmax(x)
    return vals, idx.astype(jnp.int64)


# speedup vs baseline: 2.5601x; 2.3771x over previous
import functools

import jax
import jax.numpy as jnp
from jax import lax
from jax.experimental import pallas as pl
from jax.experimental.pallas import tpu as pltpu
from jax.experimental.pallas import tpu_sc as plsc

R = 64
N = 8192

_info = plsc.get_sparse_core_info()
_NC, _NS, _L = _info.num_cores, _info.num_subcores, _info.num_lanes
_NW = _NC * _NS
_CPW = N // _NW
_NB = 2
_CW = _CPW // _NB
_GPC = _CW // _L


@functools.partial(
    pl.kernel,
    mesh=plsc.VectorSubcoreMesh(core_axis_name="c", subcore_axis_name="s"),
    out_type=(
        jax.ShapeDtypeStruct((1, N), jnp.float32),
        jax.ShapeDtypeStruct((1, N), jnp.int32),
    ),
    scratch_types=[
        pltpu.VMEM((R, _CPW), jnp.float32),
        pltpu.VMEM((_CPW,), jnp.float32),
        pltpu.VMEM((_CPW,), jnp.int32),
    ] + [pltpu.SemaphoreType.DMA] * _NB,
)
def _colmax(x_hbm, vals_hbm, idx_hbm, x_v, mv_v, mi_v, *sems):
    wid = lax.axis_index("s") * _NC + lax.axis_index("c")
    base = wid * _CPW

    copies = [
        pltpu.async_copy(
            x_hbm.at[:, pl.ds(base + c * _CW, _CW)],
            x_v.at[:, pl.ds(c * _CW, _CW)],
            sems[c],
        )
        for c in range(_NB)
    ]

    copies[0].wait()

    def group(g, carry):
        for c in range(1, _NB):
            @pl.when(g == c * _GPC)
            def _():
                copies[c].wait()

        cols = pl.ds(g * _L, _L)
        m = x_v[0, cols]
        idx = jnp.zeros((_L,), jnp.int32)
        for r in range(1, R):
            v = x_v[r, cols]
            pred = v > m
            m = jnp.where(pred, v, m)
            idx = jnp.where(pred, jnp.full((_L,), r, jnp.int32), idx)
        mv_v[cols] = m
        mi_v[cols] = idx
        return carry

    lax.fori_loop(0, _NB * _GPC, group, 0)

    pltpu.sync_copy(mv_v, vals_hbm.at[0, pl.ds(base, _CPW)])
    pltpu.sync_copy(mi_v, idx_hbm.at[0, pl.ds(base, _CPW)])


def _tc_body(x_ref, v_ref, i_ref):
    xb = x_ref[...]
    m = jnp.max(xb, axis=0, keepdims=True)
    rows = lax.broadcasted_iota(jnp.int32, xb.shape, 0)
    hit = jnp.where(xb == m, rows, R)
    i_ref[...] = jnp.min(hit, axis=0, keepdims=True)
    v_ref[...] = m


def _tc_colmax(xs, bc):
    n = xs.shape[1]
    return pl.pallas_call(
        _tc_body,
        grid=(n // bc,),
        in_specs=[pl.BlockSpec((R, bc), lambda j: (0, j))],
        out_specs=(
            pl.BlockSpec((1, bc), lambda j: (0, j)),
            pl.BlockSpec((1, bc), lambda j: (0, j)),
        ),
        out_shape=(
            jax.ShapeDtypeStruct((1, n), jnp.float32),
            jax.ShapeDtypeStruct((1, n), jnp.int32),
        ),
    )(xs)


def kernel(x):
    vals, idx = _tc_colmax(x, 512)
    return vals, idx.astype(jnp.int64)
